# Initial kernel scaffold; baseline (speedup 1.0000x reference)
import jax
import jax.numpy as jnp
from jax.experimental import pallas as pl


def kernel(hm, wh, reg):
    raise NotImplementedError("write your pallas kernel here")


# speedup vs baseline: 1.0501x; 1.0501x over previous
import jax
import jax.numpy as jnp
from jax.experimental import pallas as pl


def _nms_body(hm_ref, out_ref):
    x = hm_ref[...]
    s = jnp.clip(jax.nn.sigmoid(x), 1e-4, 1.0 - 1e-4)
    ninf = jnp.float32(-jnp.inf)
    C, H, W = s.shape
    colpad = jnp.full((C, H, 1), ninf, dtype=s.dtype)
    left = jnp.concatenate([colpad, s[:, :, :-1]], axis=2)
    right = jnp.concatenate([s[:, :, 1:], colpad], axis=2)
    rowmax = jnp.maximum(jnp.maximum(left, right), s)
    rowpad = jnp.full((C, 1, W), ninf, dtype=s.dtype)
    up = jnp.concatenate([rowpad, rowmax[:, :-1, :]], axis=1)
    down = jnp.concatenate([rowmax[:, 1:, :], rowpad], axis=1)
    hmax = jnp.maximum(jnp.maximum(up, down), rowmax)
    keep = (hmax == s).astype(s.dtype)
    out_ref[...] = s * keep


def kernel(hm, wh, reg):
    K = 100
    down_ratio = 4.0
    B, C, H, W = hm.shape
    nms = pl.pallas_call(
        _nms_body,
        out_shape=jax.ShapeDtypeStruct((C, H, W), jnp.float32),
    )(hm[0])
    scores, inds = jax.lax.top_k(nms.reshape(1, -1), K)
    clses = (inds // (H * W)).astype(jnp.int32)
    pix = inds % (H * W)
    ys = (pix // W).astype(jnp.float32)
    xs = (pix % W).astype(jnp.float32)
    reg_f = reg.reshape(B, 2, H * W)
    wh_f = wh.reshape(B, 2, H * W)
    rx = jnp.take_along_axis(reg_f[:, 0, :], pix, axis=1)
    ry = jnp.take_along_axis(reg_f[:, 1, :], pix, axis=1)
    w_ = jnp.take_along_axis(wh_f[:, 0, :], pix, axis=1)
    h_ = jnp.take_along_axis(wh_f[:, 1, :], pix, axis=1)
    xs = xs + rx
    ys = ys + ry
    bboxes = jnp.stack([xs - w_ / 2.0, ys - h_ / 2.0,
                        xs + w_ / 2.0, ys + h_ / 2.0], axis=-1)
    bboxes = bboxes * down_ratio
    det_bboxes = jnp.concatenate([bboxes, scores[..., None]], axis=-1)
    det_bboxes = det_bboxes.reshape(K, 5)
    clses_out = clses.reshape(K)
    return det_bboxes, clses_out
